# Initial kernel scaffold; baseline (speedup 1.0000x reference)
import jax
import jax.numpy as jnp
from jax.experimental import pallas as pl


def kernel(codebook, indices_pair_list):
    raise NotImplementedError("write your pallas kernel here")


# speedup vs baseline: 2.2367x; 2.2367x over previous
import jax
import jax.numpy as jnp
from jax.experimental import pallas as pl
from jax.experimental.pallas import tpu as pltpu
from jax.experimental.pallas import tpu_sc as plsc

K = 8192
D = 256
NPAIR = 16384
INV_T = 10.0
SHIFT = 10.0

BN = 1024
BI = 256
BJ = 2048
WG = 128


def _normalize_body(x_ref, f32_ref, b16_ref):
    x = x_ref[...]
    ss = jnp.sum(x * x, axis=1, keepdims=True)
    inv = 1.0 / jnp.maximum(jnp.sqrt(ss), 1e-8)
    y = x * inv
    f32_ref[...] = y
    b16_ref[...] = y.astype(jnp.bfloat16)


def _normalize(codebook):
    return pl.pallas_call(
        _normalize_body,
        grid=(K // BN,),
        in_specs=[pl.BlockSpec((BN, D), lambda i: (i, 0))],
        out_specs=[pl.BlockSpec((BN, D), lambda i: (i, 0)),
                   pl.BlockSpec((BN, D), lambda i: (i, 0))],
        out_shape=[jax.ShapeDtypeStruct((K, D), jnp.float32),
                   jax.ShapeDtypeStruct((K, D), jnp.bfloat16)],
    )(codebook)


def _lse_body(xi_ref, cb_ref, a16_ref):
    xi = xi_ref[...]

    def step(j, acc):
        cbj = cb_ref[pl.ds(j * BJ, BJ), :]
        s = jax.lax.dot_general(xi, cbj, (((1,), (1,)), ((), ())),
                                preferred_element_type=jnp.float32)
        e = jnp.exp(s * INV_T - SHIFT)
        return acc + jnp.sum(e, axis=1, keepdims=True)

    acc = jax.lax.fori_loop(0, K // BJ, step,
                            jnp.zeros((BI, 1), jnp.float32))
    a = SHIFT + jnp.log(acc)
    a16_ref[...] = jnp.broadcast_to(a, (BI, 128))


def _lse(cb16):
    return pl.pallas_call(
        _lse_body,
        grid=(K // BI,),
        in_specs=[pl.BlockSpec((BI, D), lambda i: (i, 0)),
                  pl.BlockSpec((K, D), lambda i: (0, 0))],
        out_specs=pl.BlockSpec((BI, 128), lambda i: (i, 0)),
        out_shape=jax.ShapeDtypeStruct((K, 128), jnp.float32),
    )(cb16, cb16)


def _pair_body(gi_ref, gj_ref, ai_ref, ii_ref, jj_ref, o_ref):
    s = jnp.sum(gi_ref[...] * gj_ref[...], axis=1, keepdims=True)
    x = ai_ref[:, 0:1] - s
    sp = jnp.maximum(x, 0.0) + jnp.log(1.0 + jnp.exp(-jnp.abs(x)))
    loss = jnp.where(ii_ref[...] != jj_ref[...], sp, 0.0)
    o_ref[0, 0] = jnp.sum(loss) / NPAIR


def _pair_loss(gi, gj, ai, ii2, jj2):
    return pl.pallas_call(
        _pair_body,
        out_specs=pl.BlockSpec(memory_space=pltpu.SMEM),
        out_shape=jax.ShapeDtypeStruct((1, 1), jnp.float32),
    )(gi, gj, ai, ii2, jj2)


def _sc_gather(data, idx):
    vdim = data.shape[1]
    mesh = plsc.VectorSubcoreMesh(core_axis_name="core",
                                  subcore_axis_name="subcore")

    @pl.kernel(out_type=jax.ShapeDtypeStruct((NPAIR, vdim), data.dtype),
               mesh=mesh)
    def k(d_hbm, i_hbm, o_hbm):
        def body(i_vmem, o_vmem):
            pltpu.sync_copy(d_hbm.at[i_vmem.at[0]], o_vmem)

        pltpu.emit_pipeline(
            body,
            grid=(NPAIR // WG,),
            in_specs=[pl.BlockSpec((1, WG), lambda i: (0, i))],
            out_specs=[pl.BlockSpec((WG, vdim), lambda i: (i, 0))],
            core_axis_name=("core", "subcore"),
            dimension_semantics=(pltpu.PARALLEL,),
        )(i_hbm, o_hbm)

    return k(data, idx)


def kernel(codebook, indices_pair_list):
    cbn, cb16 = _normalize(codebook)
    a16 = _lse(cb16)
    ii = indices_pair_list[:, 0].reshape(1, NPAIR)
    jj = indices_pair_list[:, 1].reshape(1, NPAIR)
    gi = _sc_gather(cbn, ii)
    gj = _sc_gather(cbn, jj)
    ai = _sc_gather(a16, ii)
    ii2 = indices_pair_list[:, 0:1]
    jj2 = indices_pair_list[:, 1:2]
    out = _pair_loss(gi, gj, ai, ii2, jj2)
    return out[0, 0]
